# baseline (speedup 1.0000x reference)
import functools

import jax
import jax.numpy as jnp
from jax import lax
from jax.experimental import pallas as pl
from jax.experimental.pallas import tpu as pltpu

_ROT = ((13, 15, 26, 6), (17, 29, 16, 24))
_K0 = 0
_K1 = 42
_K2 = _K0 ^ _K1 ^ 0x1BD11BDA
_SCHED = ((0, 1, 2, 1), (1, 2, 0, 2), (0, 0, 1, 3), (1, 1, 2, 4), (0, 2, 0, 5))

_LANES = 8192
_SUB = 512


def _gumbel_bits(flat):
    ks = (jnp.uint32(_K0), jnp.uint32(_K1), jnp.uint32(_K2))
    x0 = jnp.zeros_like(flat) + ks[0]
    x1 = flat + ks[1]
    for rset, ka, kb, inc in _SCHED:
        for r in _ROT[rset]:
            x0 = x0 + x1
            x1 = (x1 << jnp.uint32(r)) | lax.shift_right_logical(
                x1, jnp.uint32(32 - r))
            x1 = x0 ^ x1
        x0 = x0 + ks[ka]
        x1 = x1 + ks[kb] + jnp.uint32(inc)
    return x0 ^ x1


def _sampler_kernel(temps_ref, rowv_ref, logits_ref, out_ref,
                    mval, midx, *, V, C, S, NV):
    j = pl.program_id(0)

    B = mval.shape[0]
    t = temps_ref[:, :]
    safe_t = jnp.where(t == 0.0, jnp.float32(1.0), t)
    gmask = jnp.where(t == 0.0, jnp.float32(0.0), jnp.float32(1.0))
    rowv = rowv_ref[:, :]
    base = j * C
    neg_inf = jnp.float32(-jnp.inf)
    tiny = jnp.float32(jnp.finfo(jnp.float32).tiny)

    def body(k, carry):
        m, i = carry
        off = k * S
        x = logits_ref[:, pl.ds(off, S)]
        lane = lax.broadcasted_iota(jnp.int32, (B, S), 1) + (base + off)
        flat = rowv + lane.astype(jnp.uint32)
        bits = _gumbel_bits(flat)
        fb = (lax.shift_right_logical(bits, jnp.uint32(9))
              | jnp.uint32(0x3F800000))
        f = lax.bitcast_convert_type(fb, jnp.float32) - jnp.float32(1.0)
        u = jnp.maximum(tiny, f + tiny)
        g = -jnp.log(-jnp.log(u))
        val = jnp.where(lane < V, g * gmask + x / safe_t, neg_inf)
        better = val > m
        return (jnp.where(better, val, m), jnp.where(better, lane, i))

    m0 = jnp.where(j == 0, jnp.full((B, S), neg_inf, jnp.float32), mval[:])
    i0 = jnp.where(j == 0, jnp.zeros((B, S), jnp.int32), midx[:])
    m, i = lax.fori_loop(0, C // S, body, (m0, i0), unroll=False)
    mval[:] = m
    midx[:] = i

    @pl.when(j == NV - 1)
    def _fin():
        best = jnp.max(m, axis=1, keepdims=True)
        cand = jnp.where(m == best, i, jnp.int32(2**31 - 1))
        out_ref[:] = jnp.min(cand, axis=1, keepdims=True)


def kernel(logits, temperatures):
    B, V = logits.shape
    C = _LANES
    S = _SUB
    NV = pl.cdiv(V, C)
    temps = temperatures.reshape(B, 1)
    rowv = (jnp.arange(B, dtype=jnp.uint32) * jnp.uint32(V)).reshape(B, 1)
    out = pl.pallas_call(
        functools.partial(_sampler_kernel, V=V, C=C, S=_SUB, NV=NV),
        grid=(NV,),
        in_specs=[
            pl.BlockSpec((B, 1), lambda j: (0, 0)),
            pl.BlockSpec((B, 1), lambda j: (0, 0)),
            pl.BlockSpec((B, C), lambda j: (0, j)),
        ],
        out_specs=pl.BlockSpec((B, 1), lambda j: (0, 0)),
        out_shape=jax.ShapeDtypeStruct((B, 1), jnp.int32),
        scratch_shapes=[
            pltpu.VMEM((B, S), jnp.float32),
            pltpu.VMEM((B, S), jnp.int32),
        ],
    )(temps, rowv, logits)
    return out.reshape(B)
